# baseline (speedup 1.0000x reference)
import jax
import jax.numpy as jnp
from jax.experimental import pallas as pl
from jax.experimental.pallas import tpu as pltpu

_LANE = 128
_VMEM_LIMIT = 60 * 1024 * 1024
_SUBTILES = 4
_BAND = 64


def _softmax_subtiled_kernel(x_ref, o_ref, inv_ref):
    n = x_ref.shape[0]
    rows = n // _SUBTILES
    q = jax.lax.rem(pl.program_id(0), _SUBTILES)

    @pl.when(q == 0)
    def _():
        band = _BAND if n % _BAND == 0 else n
        part = None
        for b in range(0, n, band):
            s = jnp.sum(jnp.exp(x_ref[b:b + band, :]), axis=0, keepdims=True)
            part = s if part is None else part + s
        inv_ref[...] = pl.reciprocal(part, approx=False)

    xq = x_ref[pl.ds(q * rows, rows), :]
    o_ref[...] = jnp.exp(xq) * inv_ref[...]


def kernel(x):
    orig_shape = x.shape
    N = orig_shape[0]
    x2 = x.reshape(N, -1) if x.ndim != 2 else x
    D = x2.shape[1]

    budget = _VMEM_LIMIT - 8 * 1024 * 1024
    per_lane = 5 * N * 4
    tile_d = D
    for t in range(D, _LANE - 1, -_LANE):
        if D % t == 0 and t * per_lane <= budget and D // t >= 4:
            tile_d = t
            break
    else:
        for t in range(D, _LANE - 1, -_LANE):
            if D % t == 0 and t * per_lane <= budget:
                tile_d = t
                break

    subtiles = _SUBTILES if N % (_SUBTILES * 8) == 0 else 1
    n_blocks = D // tile_d
    grid = (n_blocks * subtiles,)
    rows = N // subtiles

    cost = pl.CostEstimate(
        flops=2 * N * D,
        transcendentals=N * D + N * D // subtiles,
        bytes_accessed=2 * N * D * x2.dtype.itemsize,
    )
    out = pl.pallas_call(
        _softmax_subtiled_kernel if subtiles > 1 else _softmax_whole_kernel,
        out_shape=jax.ShapeDtypeStruct((N, D), x2.dtype),
        grid=grid,
        in_specs=[pl.BlockSpec((N, tile_d), lambda j: (0, j // subtiles))],
        out_specs=pl.BlockSpec((rows, tile_d),
                               lambda j: (j % subtiles, j // subtiles)),
        scratch_shapes=[pltpu.VMEM((1, tile_d), jnp.float32)],
        compiler_params=pltpu.CompilerParams(
            dimension_semantics=("parallel",),
            vmem_limit_bytes=_VMEM_LIMIT,
        ),
        cost_estimate=cost,
    )(x2)
    return out.reshape(orig_shape)


def _softmax_whole_kernel(x_ref, o_ref, inv_ref):
    n = x_ref.shape[0]
    band = _BAND if n % _BAND == 0 else n
    part = None
    for b in range(0, n, band):
        e = jnp.exp(x_ref[b:b + band, :])
        s = jnp.sum(e, axis=0, keepdims=True)
        part = s if part is None else part + s
        o_ref[b:b + band, :] = e
    inv = pl.reciprocal(part, approx=False)
    for b in range(0, n, band):
        o_ref[b:b + band, :] = o_ref[b:b + band, :] * inv
